# Initial kernel scaffold; baseline (speedup 1.0000x reference)
import jax
import jax.numpy as jnp
from jax.experimental import pallas as pl


def kernel(x, edge_index, W_in, b_in, W_h, b_h, W_out, b_out):
    raise NotImplementedError("write your pallas kernel here")


# speedup vs baseline: 9.1296x; 9.1296x over previous
import functools

import jax
import jax.numpy as jnp
from jax import lax
from jax.experimental import pallas as pl
from jax.experimental.pallas import tpu as pltpu
from jax.experimental.pallas import tpu_sc as plsc

NC = 2
NS = 16
NW = NC * NS
CHUNK = 128
RPT = 632


def _make_deg_kernel(n_edges):
    n_iters = n_edges // (CHUNK * NW)
    n_pad = NS * RPT
    DW = 128

    mesh = plsc.VectorSubcoreMesh(core_axis_name="c", subcore_axis_name="s",
                                  num_cores=NC, num_subcores=NS)

    @functools.partial(
        pl.kernel,
        out_type=jax.ShapeDtypeStruct((NC, NS, RPT, DW), jnp.float32),
        mesh=mesh,
        scratch_types=[
            pltpu.VMEM((CHUNK,), jnp.int32),
            pltpu.VMEM((CHUNK, DW), jnp.float32),
            pltpu.VMEM_SHARED((n_pad, DW), jnp.float32),
        ],
    )
    def deg_kernel(dst_hbm, ones_hbm, zeros_hbm, out_hbm, dst_v, ones_v, acc):
        cid = lax.axis_index("c")
        sid = lax.axis_index("s")
        wid = sid * NC + cid
        pltpu.sync_copy(ones_hbm, ones_v)
        pltpu.sync_copy(zeros_hbm, acc.at[pl.ds(sid * RPT, RPT)])
        plsc.subcore_barrier()

        def body(i, carry):
            c = wid + NW * i
            pltpu.sync_copy(dst_hbm.at[pl.ds(c * CHUNK, CHUNK)], dst_v)
            pltpu.sync_copy(ones_v, acc.at[dst_v], add=True)
            return carry

        lax.fori_loop(0, n_iters, body, 0)
        plsc.subcore_barrier()
        pltpu.sync_copy(acc.at[pl.ds(sid * RPT, RPT)], out_hbm.at[cid, sid])

    return deg_kernel


def _make_agg_kernel(n_edges, d):
    n_iters = n_edges // (CHUNK * NW)
    n_pad = NS * RPT

    mesh = plsc.VectorSubcoreMesh(core_axis_name="c", subcore_axis_name="s",
                                  num_cores=NC, num_subcores=NS)

    @functools.partial(
        pl.kernel,
        out_type=jax.ShapeDtypeStruct((NC, NS, RPT, d), jnp.float32),
        mesh=mesh,
        scratch_types=[
            pltpu.VMEM((CHUNK,), jnp.int32),
            pltpu.VMEM((CHUNK,), jnp.int32),
            pltpu.VMEM((CHUNK, d), jnp.float32),
            pltpu.VMEM_SHARED((n_pad, d), jnp.float32),
            pltpu.SemaphoreType.DMA,
        ],
    )
    def agg_kernel(y_hbm, src_hbm, dst_hbm, zeros_hbm, out_hbm,
                   src_v, dst_v, rows_v, acc, sem):
        cid = lax.axis_index("c")
        sid = lax.axis_index("s")
        wid = sid * NC + cid
        pltpu.sync_copy(zeros_hbm, acc.at[pl.ds(sid * RPT, RPT)])
        plsc.subcore_barrier()

        def body(i, carry):
            base = (wid + NW * i) * CHUNK
            pltpu.sync_copy(src_hbm.at[pl.ds(base, CHUNK)], src_v)
            pltpu.sync_copy(dst_hbm.at[pl.ds(base, CHUNK)], dst_v)
            pltpu.async_copy(y_hbm.at[src_v], rows_v, sem).wait()
            pltpu.sync_copy(rows_v, acc.at[dst_v], add=True)
            return carry

        lax.fori_loop(0, n_iters, body, 0)
        plsc.subcore_barrier()
        pltpu.sync_copy(acc.at[pl.ds(sid * RPT, RPT)], out_hbm.at[cid, sid])

    return agg_kernel


_BR = 1000


def _tc_first(x, w, dinv2):
    n, d_in = x.shape
    d_out = w.shape[1]

    def body(x_ref, w_ref, dinv_ref, y_ref):
        xw = jnp.dot(x_ref[...], w_ref[...],
                     preferred_element_type=jnp.float32)
        y_ref[...] = xw * dinv_ref[...]

    return pl.pallas_call(
        body,
        grid=(n // _BR,),
        in_specs=[
            pl.BlockSpec((_BR, d_in), lambda i: (i, 0)),
            pl.BlockSpec((d_in, d_out), lambda i: (0, 0)),
            pl.BlockSpec((_BR, 1), lambda i: (i, 0)),
        ],
        out_specs=pl.BlockSpec((_BR, d_out), lambda i: (i, 0)),
        out_shape=jax.ShapeDtypeStruct((n, d_out), jnp.float32),
    )(x, w, dinv2)


def _tc_mid(p0, p1, y_prev, dinv2, b, w):
    n, d = y_prev.shape
    d_out = w.shape[1]

    def body(p0_ref, p1_ref, y_ref, dinv_ref, b_ref, w_ref, out_ref):
        agg = p0_ref[...] + p1_ref[...] + y_ref[...]
        h = jnp.maximum(agg * dinv_ref[...] + b_ref[...], 0.0)
        hw = jnp.dot(h, w_ref[...], preferred_element_type=jnp.float32)
        out_ref[...] = hw * dinv_ref[...]

    return pl.pallas_call(
        body,
        grid=(n // _BR,),
        in_specs=[
            pl.BlockSpec((_BR, d), lambda i: (i, 0)),
            pl.BlockSpec((_BR, d), lambda i: (i, 0)),
            pl.BlockSpec((_BR, d), lambda i: (i, 0)),
            pl.BlockSpec((_BR, 1), lambda i: (i, 0)),
            pl.BlockSpec((1, d), lambda i: (0, 0)),
            pl.BlockSpec((d, d_out), lambda i: (0, 0)),
        ],
        out_specs=pl.BlockSpec((_BR, d_out), lambda i: (i, 0)),
        out_shape=jax.ShapeDtypeStruct((n, d_out), jnp.float32),
    )(p0, p1, y_prev, dinv2, b, w)


def _tc_scale(p0, p1, y_prev, dinv2, b):
    n, d = y_prev.shape

    def body(p0_ref, p1_ref, y_ref, dinv_ref, b_ref, out_ref):
        agg = p0_ref[...] + p1_ref[...] + y_ref[...]
        h = jnp.maximum(agg * dinv_ref[...] + b_ref[...], 0.0)
        out_ref[...] = h * dinv_ref[...]

    return pl.pallas_call(
        body,
        grid=(n // _BR,),
        in_specs=[
            pl.BlockSpec((_BR, d), lambda i: (i, 0)),
            pl.BlockSpec((_BR, d), lambda i: (i, 0)),
            pl.BlockSpec((_BR, d), lambda i: (i, 0)),
            pl.BlockSpec((_BR, 1), lambda i: (i, 0)),
            pl.BlockSpec((1, d), lambda i: (0, 0)),
        ],
        out_specs=pl.BlockSpec((_BR, d), lambda i: (i, 0)),
        out_shape=jax.ShapeDtypeStruct((n, d), jnp.float32),
    )(p0, p1, y_prev, dinv2, b)


def _tc_final(p0, p1, t_prev, dinv2, w, b):
    n, d = t_prev.shape
    d_out = w.shape[1]

    def body(p0_ref, p1_ref, t_ref, dinv_ref, w_ref, b_ref, out_ref):
        agg = (p0_ref[...] + p1_ref[...] + t_ref[...]) * dinv_ref[...]
        z = jnp.dot(agg, w_ref[...],
                    preferred_element_type=jnp.float32) + b_ref[...]
        m = jnp.max(z, axis=1, keepdims=True)
        e = jnp.exp(z - m)
        s = jnp.sum(e, axis=1, keepdims=True)
        out_ref[...] = z - m - jnp.log(s)

    return pl.pallas_call(
        body,
        grid=(n // _BR,),
        in_specs=[
            pl.BlockSpec((_BR, d), lambda i: (i, 0)),
            pl.BlockSpec((_BR, d), lambda i: (i, 0)),
            pl.BlockSpec((_BR, d), lambda i: (i, 0)),
            pl.BlockSpec((_BR, 1), lambda i: (i, 0)),
            pl.BlockSpec((d, d_out), lambda i: (0, 0)),
            pl.BlockSpec((1, d_out), lambda i: (0, 0)),
        ],
        out_specs=pl.BlockSpec((_BR, d_out), lambda i: (i, 0)),
        out_shape=jax.ShapeDtypeStruct((n, d_out), jnp.float32),
    )(p0, p1, t_prev, dinv2, w, b)


def kernel(x, edge_index, W_in, b_in, W_h, b_h, W_out, b_out):
    n, d_in = x.shape
    n_edges = edge_index.shape[1]
    d_h = W_h.shape[0]
    d_out = W_out.shape[1]
    n_pad = NS * RPT

    ei = edge_index.astype(jnp.int32)
    e_blk = CHUNK * NW
    e_pad = ((n_edges + e_blk - 1) // e_blk) * e_blk
    src = jnp.concatenate([ei[0], jnp.zeros((e_pad - n_edges,), jnp.int32)])
    dst = jnp.concatenate(
        [ei[1], jnp.full((e_pad - n_edges,), n_pad - 1, jnp.int32)])
    ones128 = jnp.ones((CHUNK, 128), jnp.float32)
    zeros_h = jnp.zeros((RPT, d_h), jnp.float32)

    degp = _make_deg_kernel(e_pad)(dst, ones128, zeros_h)
    degp = degp.reshape(NC, n_pad, 128)
    deg = degp[0, :n, 0] + degp[1, :n, 0] + 1.0
    dinv2 = lax.rsqrt(deg)[:, None]

    agg_h = _make_agg_kernel(e_pad, d_h)

    y1 = _tc_first(x, W_in, dinv2)
    p = agg_h(y1, src, dst, zeros_h).reshape(NC, n_pad, d_h)[:, :n]
    y2 = _tc_mid(p[0], p[1], y1, dinv2, b_in.reshape(1, -1), W_h)
    p = agg_h(y2, src, dst, zeros_h).reshape(NC, n_pad, d_h)[:, :n]
    t3 = _tc_scale(p[0], p[1], y2, dinv2, b_h.reshape(1, -1))
    p = agg_h(t3, src, dst, zeros_h).reshape(NC, n_pad, d_h)[:, :n]
    return _tc_final(p[0], p[1], t3, dinv2, W_out, b_out.reshape(1, -1))
